# Initial kernel scaffold; baseline (speedup 1.0000x reference)
import jax
import jax.numpy as jnp
from jax.experimental import pallas as pl


def kernel(x_0, incidence_1, W, b):
    raise NotImplementedError("write your pallas kernel here")


# speedup vs baseline: 1.4698x; 1.4698x over previous
import jax
import jax.numpy as jnp
from jax.experimental import pallas as pl


def _pass_a(x0in_ref, inc_ref, wt_ref, b_ref, xlin_ref, x1_ref):
    @pl.when(pl.program_id(0) == 0)
    def _():
        xlin_ref[...] = (
            jnp.dot(x0in_ref[...], wt_ref[...], preferred_element_type=jnp.float32)
            + b_ref[...]
        )
    x1_ref[...] = jax.lax.dot_general(
        inc_ref[...], xlin_ref[...],
        dimension_numbers=(((0,), (0,)), ((), ())),
        preferred_element_type=jnp.float32,
    )


def _pass_b(inc_ref, x1_ref, xlin_ref, out_ref):
    acc = jnp.dot(inc_ref[...], x1_ref[...], preferred_element_type=jnp.float32)
    ns = jnp.sum(inc_ref[...], axis=1, keepdims=True)
    out_ref[...] = xlin_ref[...] + acc / ns


def kernel(x_0, incidence_1, W, b):
    n, c_in = x_0.shape
    e = incidence_1.shape[1]
    c_hid = W.shape[0]
    wt = W.T
    b2 = b.reshape(1, c_hid)

    be = min(512, e)
    xlin, x_1 = pl.pallas_call(
        _pass_a,
        grid=(pl.cdiv(e, be),),
        in_specs=[
            pl.BlockSpec((n, c_in), lambda i: (0, 0)),
            pl.BlockSpec((n, be), lambda i: (0, i)),
            pl.BlockSpec((c_in, c_hid), lambda i: (0, 0)),
            pl.BlockSpec((1, c_hid), lambda i: (0, 0)),
        ],
        out_specs=[
            pl.BlockSpec((n, c_hid), lambda i: (0, 0)),
            pl.BlockSpec((be, c_hid), lambda i: (i, 0)),
        ],
        out_shape=[
            jax.ShapeDtypeStruct((n, c_hid), jnp.float32),
            jax.ShapeDtypeStruct((e, c_hid), jnp.float32),
        ],
    )(x_0, incidence_1, wt, b2)

    bn = min(512, n)
    x0_out = pl.pallas_call(
        _pass_b,
        grid=(pl.cdiv(n, bn),),
        in_specs=[
            pl.BlockSpec((bn, e), lambda i: (i, 0)),
            pl.BlockSpec((e, c_hid), lambda i: (0, 0)),
            pl.BlockSpec((bn, c_hid), lambda i: (i, 0)),
        ],
        out_specs=pl.BlockSpec((bn, c_hid), lambda i: (i, 0)),
        out_shape=jax.ShapeDtypeStruct((n, c_hid), jnp.float32),
    )(incidence_1, x_1, xlin)

    return (x0_out, x_1)
